# baseline (speedup 1.0000x reference)
import functools

import jax
import jax.numpy as jnp
from jax.experimental import pallas as pl
from jax.experimental.pallas import tpu as pltpu
from jax.experimental.pallas import tpu_sc as plsc

DIM = 768
CODE_DIM = 256
CODEBOOK = 8192
NTOK = 16384
TILE = 512
PROJ_TILE = 1024
NSPLIT = 4
GATHER_WINDOW = 64


NCHUNK = 8
CW = CODEBOOK // NCHUNK


def _e2_body(et_ref, o_ref):
    o_ref[...] = jnp.sum(et_ref[...] * et_ref[...], axis=0, keepdims=True)


def _e2_call(et):
    return pl.pallas_call(
        _e2_body,
        out_shape=jax.ShapeDtypeStruct((1, CODEBOOK), jnp.float32),
    )(et)


def _argmin_body(x_ref, w_ref, b_ref, et_ref, e2_ref, idx_ref, loss_ref):
    i = pl.program_id(0)
    xp = jnp.dot(x_ref[...], w_ref[...],
                 preferred_element_type=jnp.float32) + b_ref[...]
    f2 = jnp.sum(xp * xp, axis=1, keepdims=True)
    q = (f2
         - 2.0 * jnp.dot(xp, et_ref[...], preferred_element_type=jnp.float32)
         + e2_ref[...])
    idx_ref[...] = jnp.argmin(q, axis=1).astype(jnp.int32)
    part = jnp.sum(jnp.min(q, axis=1))[None, None]

    @pl.when(i == 0)
    def _():
        loss_ref[...] = jnp.zeros_like(loss_ref)

    loss_ref[...] = loss_ref[...] + part


def _argmin_call(xs, w_in, b_in, et, e2):
    ntok = xs.shape[0]
    grid = ntok // TILE
    return pl.pallas_call(
        _argmin_body,
        grid=(grid,),
        in_specs=[
            pl.BlockSpec((TILE, DIM), lambda i: (i, 0)),
            pl.BlockSpec((DIM, CODE_DIM), lambda i: (0, 0)),
            pl.BlockSpec((1, CODE_DIM), lambda i: (0, 0)),
            pl.BlockSpec((CODE_DIM, CODEBOOK), lambda i: (0, 0)),
            pl.BlockSpec((1, CODEBOOK), lambda i: (0, 0)),
        ],
        out_specs=[
            pl.BlockSpec((TILE,), lambda i: (i,)),
            pl.BlockSpec((1, 1), lambda i: (0, 0)),
        ],
        out_shape=[
            jax.ShapeDtypeStruct((ntok,), jnp.int32),
            jax.ShapeDtypeStruct((1, 1), jnp.float32),
        ],
    )(xs, w_in, b_in, et, e2)


def _proj_body(e_ref, w_ref, b_ref, o_ref):
    o_ref[...] = jnp.dot(e_ref[...], w_ref[...],
                         preferred_element_type=jnp.float32) + b_ref[...]


def _proj_call(embed, w_out, b_out):
    grid = CODEBOOK // PROJ_TILE
    return pl.pallas_call(
        _proj_body,
        grid=(grid,),
        in_specs=[
            pl.BlockSpec((PROJ_TILE, CODE_DIM), lambda i: (i, 0)),
            pl.BlockSpec((CODE_DIM, DIM), lambda i: (0, 0)),
            pl.BlockSpec((1, DIM), lambda i: (0, 0)),
        ],
        out_specs=pl.BlockSpec((PROJ_TILE, DIM), lambda i: (i, 0)),
        out_shape=jax.ShapeDtypeStruct((CODEBOOK, DIM), jnp.float32),
    )(embed, w_out, b_out)


def _sc_gather(table, idx):
    ntok = idx.shape[0]
    mesh = plsc.VectorSubcoreMesh(core_axis_name="core",
                                  subcore_axis_name="subcore")
    info = plsc.get_sparse_core_info()
    nw = info.num_cores * info.num_subcores
    b_per_w = ntok // nw
    chunk = min(GATHER_WINDOW, b_per_w)
    nchunks = b_per_w // chunk

    @functools.partial(
        pl.kernel,
        out_type=jax.ShapeDtypeStruct((ntok, DIM), jnp.float32),
        mesh=mesh,
        scratch_types=[
            pltpu.VMEM((b_per_w,), jnp.int32),
            pltpu.VMEM((chunk, DIM), jnp.float32),
            pltpu.VMEM((chunk, DIM), jnp.float32),
            pltpu.SemaphoreType.DMA,
            pltpu.SemaphoreType.DMA,
        ])
    def k(table_hbm, idx_hbm, out_hbm, idx_v, rows0, rows1, gsem, wsem):
        wid = (jax.lax.axis_index("subcore") * info.num_cores
               + jax.lax.axis_index("core"))
        base = wid * b_per_w
        pltpu.sync_copy(idx_hbm.at[pl.ds(base, b_per_w)], idx_v)
        bufs = (rows0, rows1)

        def gather_start(j):
            return pltpu.async_copy(
                table_hbm.at[idx_v.at[pl.ds(j * chunk, chunk)]],
                bufs[j % 2], gsem)

        def write_start(j):
            return pltpu.async_copy(
                bufs[j % 2], out_hbm.at[pl.ds(base + j * chunk, chunk)], wsem)

        gathers = {0: gather_start(0)}
        writes = {}
        for j in range(nchunks):
            if j + 1 < nchunks:
                if j - 1 >= 0:
                    writes[j - 1].wait()
                gathers[j + 1] = gather_start(j + 1)
            gathers[j].wait()
            writes[j] = write_start(j)
        if nchunks >= 2:
            writes[nchunks - 2].wait()
        writes[nchunks - 1].wait()

    return k(table, idx)


def kernel(x, W_in, b_in, embed, W_out, b_out):
    xf = x.reshape(NTOK, DIM)
    et = embed.T
    e2 = _e2_call(et)
    proj = _proj_call(embed, W_out, b_out.reshape(1, DIM))
    b_in2 = b_in.reshape(1, CODE_DIM)
    tok_s = NTOK // NSPLIT
    outs, idxs, loss_total = [], [], None
    for s in range(NSPLIT):
        xs = jax.lax.slice(xf, (s * tok_s, 0), ((s + 1) * tok_s, DIM))
        idx_s, loss_s = _argmin_call(xs, W_in, b_in2, et, e2)
        outs.append(_sc_gather(proj, idx_s))
        idxs.append(idx_s)
        loss_total = loss_s if loss_total is None else loss_total + loss_s
    outf = jnp.concatenate(outs, axis=0)
    idx = jnp.concatenate(idxs, axis=0)
    loss = loss_total[0, 0] / (NTOK * CODE_DIM)
    return (outf.reshape(x.shape[0], x.shape[1], DIM),
            idx.reshape(x.shape[0], x.shape[1]),
            loss)


# speedup vs baseline: 1.3070x; 1.3070x over previous
import functools

import jax
import jax.numpy as jnp
from jax.experimental import pallas as pl
from jax.experimental.pallas import tpu as pltpu
from jax.experimental.pallas import tpu_sc as plsc

DIM = 768
CODE_DIM = 256
CODEBOOK = 8192
NTOK = 16384
TILE = 512
PROJ_TILE = 1024
NSPLIT = 4
GATHER_WINDOW = 64


NCHUNK = 8
CW = CODEBOOK // NCHUNK


def _e2_body(et_ref, o_ref):
    o_ref[...] = jnp.sum(et_ref[...] * et_ref[...], axis=0, keepdims=True)


def _e2_call(et):
    return pl.pallas_call(
        _e2_body,
        out_shape=jax.ShapeDtypeStruct((1, CODEBOOK), jnp.float32),
    )(et)


def _argmin_body(x_ref, w_ref, b_ref, et_ref, e2_ref, idx_ref, loss_ref):
    i = pl.program_id(0)
    xp = jnp.dot(x_ref[...], w_ref[...],
                 preferred_element_type=jnp.float32) + b_ref[...]
    f2 = jnp.sum(xp * xp, axis=1, keepdims=True)
    q = (f2
         - 2.0 * jnp.dot(xp, et_ref[...], preferred_element_type=jnp.float32)
         + e2_ref[...])
    idx_ref[...] = jnp.argmin(q, axis=1).astype(jnp.int32)
    part = jnp.sum(jnp.min(q, axis=1))[None, None]

    @pl.when(i == 0)
    def _():
        loss_ref[...] = jnp.zeros_like(loss_ref)

    loss_ref[...] = loss_ref[...] + part


def _argmin_call(xf, w_in, b_in, et, e2, tok_off, tok_s):
    grid = tok_s // TILE
    off = tok_off // TILE
    return pl.pallas_call(
        _argmin_body,
        grid=(grid,),
        in_specs=[
            pl.BlockSpec((TILE, DIM), lambda i: (i + off, 0)),
            pl.BlockSpec((DIM, CODE_DIM), lambda i: (0, 0)),
            pl.BlockSpec((1, CODE_DIM), lambda i: (0, 0)),
            pl.BlockSpec((CODE_DIM, CODEBOOK), lambda i: (0, 0)),
            pl.BlockSpec((1, CODEBOOK), lambda i: (0, 0)),
        ],
        out_specs=[
            pl.BlockSpec((TILE,), lambda i: (i,)),
            pl.BlockSpec((1, 1), lambda i: (0, 0)),
        ],
        out_shape=[
            jax.ShapeDtypeStruct((tok_s,), jnp.int32),
            jax.ShapeDtypeStruct((1, 1), jnp.float32),
        ],
    )(xf, w_in, b_in, et, e2)


def _proj_body(e_ref, w_ref, b_ref, o_ref):
    o_ref[...] = jnp.dot(e_ref[...], w_ref[...],
                         preferred_element_type=jnp.float32) + b_ref[...]


def _proj_call(embed, w_out, b_out):
    grid = CODEBOOK // PROJ_TILE
    return pl.pallas_call(
        _proj_body,
        grid=(grid,),
        in_specs=[
            pl.BlockSpec((PROJ_TILE, CODE_DIM), lambda i: (i, 0)),
            pl.BlockSpec((CODE_DIM, DIM), lambda i: (0, 0)),
            pl.BlockSpec((1, DIM), lambda i: (0, 0)),
        ],
        out_specs=pl.BlockSpec((PROJ_TILE, DIM), lambda i: (i, 0)),
        out_shape=jax.ShapeDtypeStruct((CODEBOOK, DIM), jnp.float32),
    )(embed, w_out, b_out)


def _sc_gather(table, idx, tok_s, out_rows, row_off):
    mesh = plsc.VectorSubcoreMesh(core_axis_name="core",
                                  subcore_axis_name="subcore")
    info = plsc.get_sparse_core_info()
    nw = info.num_cores * info.num_subcores
    b_per_w = tok_s // nw
    chunk = 32
    nchunks = b_per_w // chunk

    @functools.partial(
        pl.kernel,
        out_type=jax.ShapeDtypeStruct((out_rows, DIM), jnp.float32),
        mesh=mesh,
        scratch_types=[
            pltpu.VMEM((b_per_w,), jnp.int32),
            pltpu.VMEM((chunk, DIM), jnp.float32),
            pltpu.VMEM((chunk, DIM), jnp.float32),
            pltpu.SemaphoreType.DMA,
            pltpu.SemaphoreType.DMA,
        ])
    def k(table_hbm, idx_hbm, out_hbm, idx_v, rows0, rows1, gsem, wsem):
        wid = (jax.lax.axis_index("subcore") * info.num_cores
               + jax.lax.axis_index("core"))
        base = wid * b_per_w
        pltpu.sync_copy(idx_hbm.at[pl.ds(base, b_per_w)], idx_v)
        bufs = (rows0, rows1)

        def gather_start(j):
            return pltpu.async_copy(
                table_hbm.at[idx_v.at[pl.ds(j * chunk, chunk)]],
                bufs[j % 2], gsem)

        def write_start(j):
            return pltpu.async_copy(
                bufs[j % 2],
                out_hbm.at[pl.ds(row_off + base + j * chunk, chunk)], wsem)

        gathers = {0: gather_start(0)}
        writes = {}
        for j in range(nchunks):
            if j + 1 < nchunks:
                if j - 1 >= 0:
                    writes[j - 1].wait()
                gathers[j + 1] = gather_start(j + 1)
            gathers[j].wait()
            writes[j] = write_start(j)
        if nchunks >= 2:
            writes[nchunks - 2].wait()
        writes[nchunks - 1].wait()

    return k(table, idx)


def kernel(x, W_in, b_in, embed, W_out, b_out):
    xf = x.reshape(NTOK, DIM)
    et = embed.T
    e2 = _e2_call(et)
    proj = _proj_call(embed, W_out, b_out.reshape(1, DIM))
    b_in2 = b_in.reshape(1, CODE_DIM)
    sizes = (5120, 4096, 4096, 3072)
    outf = None
    idxs, loss_total = [], None
    off = 0
    for s, tok_s in enumerate(sizes):
        idx_s, loss_s = _argmin_call(xf, W_in, b_in2, et, e2, off, tok_s)
        idxs.append(idx_s)
        loss_total = loss_s if loss_total is None else loss_total + loss_s
        if s == 0:
            outf = _sc_gather(proj, idx_s, tok_s, NTOK, 0)
        else:
            out_s = _sc_gather(proj, idx_s, tok_s, tok_s, 0)
            outf = jax.lax.dynamic_update_slice(outf, out_s, (off, 0))
        off += tok_s
    idx = jnp.concatenate(idxs, axis=0)
    loss = loss_total[0, 0] / (NTOK * CODE_DIM)
    return (outf.reshape(x.shape[0], x.shape[1], DIM),
            idx.reshape(x.shape[0], x.shape[1]),
            loss)
